# Initial kernel scaffold; baseline (speedup 1.0000x reference)
import jax
import jax.numpy as jnp
from jax.experimental import pallas as pl


def kernel(inputs, shared_weights, map_weights):
    raise NotImplementedError("write your pallas kernel here")


# speedup vs baseline: 2.5487x; 2.5487x over previous
import functools

import jax
import jax.numpy as jnp
from jax import lax
from jax.experimental import pallas as pl
from jax.experimental.pallas import tpu as pltpu
from jax.experimental.pallas import tpu_sc as plsc

VOCAB = 100000
EMB = 128
HID = 1024
SCALE = float(EMB) ** 0.5


_NW = 32
_CHUNK = 800


def _sc_gather_body(table_hbm, idx_hbm, out_hbm, idx_v, rows_v, sem,
                    *, n_tokens):
    b_per_w = n_tokens // _NW
    n_chunks = b_per_w // _CHUNK
    wid = lax.axis_index("s") * 2 + lax.axis_index("c")
    base = wid * b_per_w

    def chunk(i, carry):
        start = base + i * _CHUNK
        pltpu.sync_copy(idx_hbm.at[pl.ds(start, _CHUNK)], idx_v)
        pltpu.async_copy(table_hbm.at[idx_v], rows_v, sem).wait()
        pltpu.sync_copy(rows_v, out_hbm.at[pl.ds(start, _CHUNK)])
        return carry

    lax.fori_loop(0, n_chunks, chunk, 0)


def _sc_gather(table, idx_flat):
    n_tokens = idx_flat.shape[0]
    mesh = plsc.VectorSubcoreMesh(core_axis_name="c", subcore_axis_name="s")
    return pl.kernel(
        functools.partial(_sc_gather_body, n_tokens=n_tokens),
        out_type=jax.ShapeDtypeStruct((n_tokens, EMB), jnp.float32),
        mesh=mesh,
        scratch_types=[
            pltpu.VMEM((_CHUNK,), jnp.int32),
            pltpu.VMEM((_CHUNK, EMB), jnp.float32),
            pltpu.SemaphoreType.DMA,
        ],
    )(table, idx_flat)


_TOK_BLK = 512


def _tc_project_body(emb_ref, ids_ref, w_ref, out_ref):
    mask = ids_ref[...] != 0
    e = jnp.where(mask, emb_ref[...], 0.0) * SCALE
    out_ref[...] = jnp.dot(e, w_ref[...], preferred_element_type=jnp.float32)


def _tc_project(gathered, ids_col, w):
    n_tokens = gathered.shape[0]
    grid = (n_tokens // _TOK_BLK,)
    return pl.pallas_call(
        _tc_project_body,
        grid=grid,
        in_specs=[
            pl.BlockSpec((_TOK_BLK, EMB), lambda i: (i, 0)),
            pl.BlockSpec((_TOK_BLK, 1), lambda i: (i, 0)),
            pl.BlockSpec((EMB, HID), lambda i: (0, 0)),
        ],
        out_specs=pl.BlockSpec((_TOK_BLK, HID), lambda i: (i, 0)),
        out_shape=jax.ShapeDtypeStruct((n_tokens, HID), jnp.float32),
    )(gathered, ids_col, w)


def kernel(inputs, shared_weights, map_weights):
    b, l = inputs.shape
    idx_flat = inputs.reshape(-1)
    gathered = _sc_gather(shared_weights, idx_flat)
    out2d = _tc_project(gathered, idx_flat.reshape(-1, 1), map_weights)
    return out2d.reshape(b, l, HID)


# speedup vs baseline: 3.4091x; 1.3375x over previous
import functools

import jax
import jax.numpy as jnp
from jax import lax
from jax.experimental import pallas as pl
from jax.experimental.pallas import tpu as pltpu
from jax.experimental.pallas import tpu_sc as plsc

VOCAB = 100000
EMB = 128
HID = 1024
SCALE = float(EMB) ** 0.5


_NW = 32
_CHUNK = 800


def _sc_gather_body(table_hbm, idx_hbm, out_hbm, idx_v, rows_v, sem,
                    *, n_tokens):
    b_per_w = n_tokens // _NW
    n_chunks = b_per_w // _CHUNK
    wid = lax.axis_index("s") * 2 + lax.axis_index("c")
    base = wid * b_per_w

    def chunk(i, carry):
        start = base + i * _CHUNK
        pltpu.sync_copy(idx_hbm.at[pl.ds(start, _CHUNK)], idx_v)
        pltpu.async_copy(table_hbm.at[idx_v], rows_v, sem).wait()
        pltpu.sync_copy(rows_v, out_hbm.at[pl.ds(start, _CHUNK)])
        return carry

    lax.fori_loop(0, n_chunks, chunk, 0)


def _sc_gather(table, idx_flat):
    n_tokens = idx_flat.shape[0]
    mesh = plsc.VectorSubcoreMesh(core_axis_name="c", subcore_axis_name="s")
    return pl.kernel(
        functools.partial(_sc_gather_body, n_tokens=n_tokens),
        out_type=jax.ShapeDtypeStruct((n_tokens, EMB), jnp.float32),
        mesh=mesh,
        scratch_types=[
            pltpu.VMEM((_CHUNK,), jnp.int32),
            pltpu.VMEM((_CHUNK, EMB), jnp.float32),
            pltpu.SemaphoreType.DMA,
        ],
    )(table, idx_flat)


_TOK_BLK = 2048


def _tc_project_body(emb_ref, ids_ref, w_ref, out_ref):
    mask = ids_ref[...] != 0
    e = jnp.where(mask, emb_ref[...], 0.0) * SCALE
    out_ref[...] = jnp.dot(e, w_ref[...], preferred_element_type=jnp.float32)


def _tc_project(gathered, ids_col, w):
    n_tokens = gathered.shape[0]
    grid = (n_tokens // _TOK_BLK,)
    return pl.pallas_call(
        _tc_project_body,
        grid=grid,
        in_specs=[
            pl.BlockSpec((_TOK_BLK, EMB), lambda i: (i, 0)),
            pl.BlockSpec((_TOK_BLK, 1), lambda i: (i, 0)),
            pl.BlockSpec((EMB, HID), lambda i: (0, 0)),
        ],
        out_specs=pl.BlockSpec((_TOK_BLK, HID), lambda i: (i, 0)),
        out_shape=jax.ShapeDtypeStruct((n_tokens, HID), jnp.float32),
    )(gathered, ids_col, w)


def kernel(inputs, shared_weights, map_weights):
    b, l = inputs.shape
    idx_flat = inputs.reshape(-1)
    gathered = _sc_gather(shared_weights, idx_flat)
    out2d = _tc_project(gathered, idx_flat.reshape(-1, 1), map_weights)
    return out2d.reshape(b, l, HID)


# speedup vs baseline: 3.4710x; 1.0182x over previous
import functools

import jax
import jax.numpy as jnp
from jax import lax
from jax.experimental import pallas as pl
from jax.experimental.pallas import tpu as pltpu
from jax.experimental.pallas import tpu_sc as plsc

VOCAB = 100000
EMB = 128
HID = 1024
SCALE = float(EMB) ** 0.5


_NW = 32
_CHUNK = 800


def _sc_gather_body(table_hbm, idx_hbm, out_hbm, idx_v, rows_v, sem,
                    *, n_tokens):
    b_per_w = n_tokens // _NW
    n_chunks = b_per_w // _CHUNK
    wid = lax.axis_index("s") * 2 + lax.axis_index("c")
    base = wid * b_per_w

    def chunk(i, carry):
        start = base + i * _CHUNK
        pltpu.sync_copy(idx_hbm.at[pl.ds(start, _CHUNK)], idx_v)
        pltpu.async_copy(table_hbm.at[idx_v], rows_v, sem).wait()
        pltpu.sync_copy(rows_v, out_hbm.at[pl.ds(start, _CHUNK)])
        return carry

    lax.fori_loop(0, n_chunks, chunk, 0)


def _sc_gather(table, idx_flat):
    n_tokens = idx_flat.shape[0]
    mesh = plsc.VectorSubcoreMesh(core_axis_name="c", subcore_axis_name="s")
    return pl.kernel(
        functools.partial(_sc_gather_body, n_tokens=n_tokens),
        out_type=jax.ShapeDtypeStruct((n_tokens, EMB), jnp.float32),
        mesh=mesh,
        scratch_types=[
            pltpu.VMEM((_CHUNK,), jnp.int32),
            pltpu.VMEM((_CHUNK, EMB), jnp.float32),
            pltpu.SemaphoreType.DMA,
        ],
    )(table, idx_flat)


_TOK_BLK = 4096


def _tc_project_body(emb_ref, ids_ref, w_ref, out_ref):
    mask = ids_ref[...] != 0
    e = jnp.where(mask, emb_ref[...], 0.0) * SCALE
    out_ref[...] = jnp.dot(e, w_ref[...], preferred_element_type=jnp.float32)


def _tc_project(gathered, ids_col, w):
    n_tokens = gathered.shape[0]
    grid = (n_tokens // _TOK_BLK,)
    return pl.pallas_call(
        _tc_project_body,
        grid=grid,
        in_specs=[
            pl.BlockSpec((_TOK_BLK, EMB), lambda i: (i, 0)),
            pl.BlockSpec((_TOK_BLK, 1), lambda i: (i, 0)),
            pl.BlockSpec((EMB, HID), lambda i: (0, 0)),
        ],
        out_specs=pl.BlockSpec((_TOK_BLK, HID), lambda i: (i, 0)),
        out_shape=jax.ShapeDtypeStruct((n_tokens, HID), jnp.float32),
    )(gathered, ids_col, w)


def kernel(inputs, shared_weights, map_weights):
    b, l = inputs.shape
    idx_flat = inputs.reshape(-1)
    gathered = _sc_gather(shared_weights, idx_flat)
    out2d = _tc_project(gathered, idx_flat.reshape(-1, 1), map_weights)
    return out2d.reshape(b, l, HID)
